# baseline (speedup 1.0000x reference)
import functools

import jax
import jax.numpy as jnp
from jax import lax
from jax.experimental import pallas as pl
from jax.experimental.pallas import tpu as pltpu
from jax.experimental.pallas import tpu_sc as plsc

_B = 4
_L = 8192
_DH = 64
_N = _B * _L
_NC = 2
_NS = 16
_NW = _NC * _NS
_CH = 128
_NCH = _N // (_NW * _CH)
_T = 2 * _NCH
_K = 12
_G = 6

_mesh = plsc.VectorSubcoreMesh(core_axis_name="c", subcore_axis_name="s")


@functools.partial(
    pl.kernel,
    mesh=_mesh,
    compiler_params=pltpu.CompilerParams(use_tc_tiling_on_sc=False),
    out_type=(
        jax.ShapeDtypeStruct((_N // _CH, _CH, _DH), jnp.float32),
        jax.ShapeDtypeStruct((_N // _CH, _CH, _DH), jnp.float32),
    ),
    scratch_types=[
        pltpu.VMEM((_NCH, _CH), jnp.int32),
        pltpu.VMEM((_K, _CH, _DH), jnp.float32),
        pltpu.SemaphoreType.DMA((_K,)),
        pltpu.SemaphoreType.DMA((_K,)),
    ],
)
def _rope_gather(pos_hbm, cos_hbm, sin_hbm, cos_out, sin_out,
                 idx_v, rows_v, sem_g, sem_o):
    wid = lax.axis_index("s") * _NC + lax.axis_index("c")
    base = wid * _NCH
    pltpu.sync_copy(pos_hbm.at[pl.ds(base, _NCH)], idx_v)

    def fire_gather(t):
        tbl = cos_hbm if t < _NCH else sin_hbm
        j = t % _NCH
        s = t % _K
        return pltpu.async_copy(tbl.at[idx_v.at[j]], rows_v.at[s],
                                sem_g.at[s])

    def fire_out(t):
        out = cos_out if t < _NCH else sin_out
        j = t % _NCH
        s = t % _K
        return pltpu.async_copy(rows_v.at[s], out.at[base + j],
                                sem_o.at[s])

    g = [None] * _T
    o = [None] * _T
    for t in range(_G):
        g[t] = fire_gather(t)
    for t in range(_T):
        g[t].wait()
        o[t] = fire_out(t)
        nt = t + _G
        if nt < _T:
            if nt >= _K:
                o[nt - _K].wait()
            g[nt] = fire_gather(nt)
    for t in range(max(0, _T - _K), _T):
        o[t].wait()


def kernel(positions, cos_cached, sin_cached):
    b, l = positions.shape
    msl, _, dh = cos_cached.shape
    pos = positions.reshape(_N // _CH, _CH)
    cos_t = cos_cached.reshape(msl, dh)
    sin_t = sin_cached.reshape(msl, dh)
    cos_o, sin_o = _rope_gather(pos, cos_t, sin_t)
    return (cos_o.reshape(b, l, 1, dh), sin_o.reshape(b, l, 1, dh))
